# baseline (speedup 1.0000x reference)
import functools

import jax
import jax.numpy as jnp
from jax import lax
from jax.experimental import pallas as pl
from jax.experimental.pallas import tpu as pltpu
from jax.experimental.pallas import tpu_sc as plsc

_LAMADA = 0.001
_B = 16384
_D = 64
_NC = 2
_NS = 16
_NW = _NC * _NS
_BPW = _B // _NW
_CHUNK = 128
_NCHUNK = _BPW // _CHUNK
_UNROLL = 16


def _tile_body(user0_hbm, item0_hbm, ratings_hbm, euser_hbm, eitem_hbm,
               out_hbm, idx_u, idx_i, urows, irows, rat, res, sem):
    wid = lax.axis_index("s") * _NC + lax.axis_index("c")
    base = wid * _BPW

    pltpu.sync_copy(user0_hbm.at[pl.ds(base, _BPW)], idx_u)
    pltpu.sync_copy(item0_hbm.at[pl.ds(base, _BPW)], idx_i)
    pltpu.sync_copy(ratings_hbm.at[pl.ds(base, _BPW)], rat)

    copies = []
    for j in range(_NCHUNK):
        sl = pl.ds(j * _CHUNK, _CHUNK)
        copies.append(pltpu.async_copy(euser_hbm.at[idx_u.at[sl]], urows.at[sl], sem))
        copies.append(pltpu.async_copy(eitem_hbm.at[idx_i.at[sl]], irows.at[sl], sem))
    for c in copies:
        c.wait()

    lane = lax.iota(jnp.int32, 16)
    zeros = jnp.zeros((16,), jnp.float32)

    def row_block(blk, carry):
        loss2_acc, u2_acc, i2_acc = carry
        rv = rat[pl.ds(blk * _UNROLL, 16)]
        rowv = blk * _UNROLL + lane
        acc_p = zeros
        au = zeros
        ai = zeros
        col = jnp.zeros((16,), jnp.int32)
        one = jnp.full((16,), 1, jnp.int32)
        for d in range(_D):
            pu = plsc.load_gather(urows, [rowv, col])
            pi = plsc.load_gather(irows, [rowv, col])
            acc_p = acc_p + pu * pi
            au = au + pu * pu
            ai = ai + pi * pi
            if d != _D - 1:
                col = col + one
        err = acc_p - rv
        return (loss2_acc + err * err, u2_acc + au, i2_acc + ai)

    loss2_acc, u2_acc, i2_acc = lax.fori_loop(
        0, _BPW // _UNROLL, row_block, (zeros, zeros, zeros))

    loss2_s = jnp.sum(loss2_acc) * (1.0 / _B)
    u2_s = jnp.sum(u2_acc) * (_LAMADA / (_B * _D))
    i2_s = jnp.sum(i2_acc) * (_LAMADA / (_B * _D))

    vec = (jnp.where(lane == 0, jnp.full((16,), loss2_s), zeros)
           + jnp.where(lane == 1, jnp.full((16,), u2_s), zeros)
           + jnp.where(lane == 2, jnp.full((16,), i2_s), zeros))
    res[...] = vec
    pltpu.sync_copy(res, out_hbm.at[wid])


@jax.jit
def _bpr_partials(user0, item_i0, ratings, embed_user, embed_item):
    mesh = plsc.VectorSubcoreMesh(core_axis_name="c", subcore_axis_name="s")
    kfn = functools.partial(
        pl.kernel,
        out_type=jax.ShapeDtypeStruct((_NW, 16), jnp.float32),
        mesh=mesh,
        compiler_params=pltpu.CompilerParams(
            needs_layout_passes=False, use_tc_tiling_on_sc=False),
        scratch_types=[
            pltpu.VMEM((_BPW,), jnp.int32),
            pltpu.VMEM((_BPW,), jnp.int32),
            pltpu.VMEM((_BPW, _D), jnp.float32),
            pltpu.VMEM((_BPW, _D), jnp.float32),
            pltpu.VMEM((_BPW,), jnp.float32),
            pltpu.VMEM((16,), jnp.float32),
            pltpu.SemaphoreType.DMA,
        ],
    )(_tile_body)
    return kfn(user0, item_i0, ratings, embed_user, embed_item)


_RB = 8192


def _transpose_body(ut_ref, it_ref, uo_ref, io_ref):
    eye = (lax.broadcasted_iota(jnp.int32, (_D, _D), 0)
           == lax.broadcasted_iota(jnp.int32, (_D, _D), 1)).astype(jnp.float32)
    dn = (((0,), (0,)), ((), ()))
    uo_ref[...] = lax.dot_general(ut_ref[...], eye, dn,
                                  preferred_element_type=jnp.float32)
    io_ref[...] = lax.dot_general(it_ref[...], eye, dn,
                                  preferred_element_type=jnp.float32)


def _relayout_tables(embed_user, embed_item):
    eu_t = embed_user.T
    ei_t = embed_item.T
    n = eu_t.shape[1]
    return pl.pallas_call(
        _transpose_body,
        grid=(pl.cdiv(n, _RB),),
        in_specs=[
            pl.BlockSpec((_D, _RB), lambda j: (0, j)),
            pl.BlockSpec((_D, _RB), lambda j: (0, j)),
        ],
        out_specs=[
            pl.BlockSpec((_RB, _D), lambda j: (j, 0)),
            pl.BlockSpec((_RB, _D), lambda j: (j, 0)),
        ],
        out_shape=[
            jax.ShapeDtypeStruct((n, _D), jnp.float32),
            jax.ShapeDtypeStruct((n, _D), jnp.float32),
        ],
    )(eu_t, ei_t)


def kernel(user0, item_i0, ratings, embed_user, embed_item):
    eu_rm, ei_rm = _relayout_tables(embed_user, embed_item)
    parts = _bpr_partials(user0, item_i0, ratings, eu_rm, ei_rm)
    s = jnp.sum(parts, axis=0)
    loss2 = s[0]
    l2 = s[1] + s[2]
    return (loss2 + l2, loss2, l2)


# speedup vs baseline: 1.2078x; 1.2078x over previous
import functools

import jax
import jax.numpy as jnp
from jax import lax
from jax.experimental import pallas as pl
from jax.experimental.pallas import tpu as pltpu
from jax.experimental.pallas import tpu_sc as plsc

_LAMADA = 0.001
_B = 16384
_D = 64
_NC = 2
_NS = 16
_NW = _NC * _NS
_BPW = _B // _NW
_CHUNK = 128
_NCHUNK = _BPW // _CHUNK
_UNROLL = 16


def _tile_body(user0_hbm, item0_hbm, ratings_hbm, euser_hbm, eitem_hbm,
               out_hbm, idx_u, idx_i, urows, irows, rat, res, sem):
    wid = lax.axis_index("s") * _NC + lax.axis_index("c")
    base = wid * _BPW

    pltpu.sync_copy(user0_hbm.at[pl.ds(base, _BPW)], idx_u)
    pltpu.sync_copy(item0_hbm.at[pl.ds(base, _BPW)], idx_i)
    pltpu.sync_copy(ratings_hbm.at[pl.ds(base, _BPW)], rat)

    copies = []
    for j in range(_NCHUNK):
        sl = pl.ds(j * _CHUNK, _CHUNK)
        copies.append(pltpu.async_copy(euser_hbm.at[idx_u.at[sl]], urows.at[sl], sem))
        copies.append(pltpu.async_copy(eitem_hbm.at[idx_i.at[sl]], irows.at[sl], sem))
    for c in copies:
        c.wait()

    lane = lax.iota(jnp.int32, 16)
    zeros = jnp.zeros((16,), jnp.float32)

    def row_block(blk, carry):
        loss2_acc, u2_acc, i2_acc = carry
        rv = rat[pl.ds(blk * _UNROLL, 16)]
        for k in range(_UNROLL):
            r = blk * _UNROLL + k
            us = [urows[r, pl.ds(16 * c, 16)] for c in range(_D // 16)]
            vs = [irows[r, pl.ds(16 * c, 16)] for c in range(_D // 16)]
            t = us[0] * vs[0]
            for c in range(1, _D // 16):
                t = t + us[c] * vs[c]
            err = jnp.sum(t) - rv[k]
            loss2_acc = loss2_acc + err * err
            for c in range(_D // 16):
                u2_acc = u2_acc + us[c] * us[c]
                i2_acc = i2_acc + vs[c] * vs[c]
        return (loss2_acc, u2_acc, i2_acc)

    loss2_s, u2_acc, i2_acc = lax.fori_loop(
        0, _BPW // _UNROLL, row_block, (jnp.float32(0.0), zeros, zeros))

    loss2_s = loss2_s * (1.0 / _B)
    u2_s = jnp.sum(u2_acc) * (_LAMADA / (_B * _D))
    i2_s = jnp.sum(i2_acc) * (_LAMADA / (_B * _D))

    vec = (jnp.where(lane == 0, jnp.full((16,), loss2_s), zeros)
           + jnp.where(lane == 1, jnp.full((16,), u2_s), zeros)
           + jnp.where(lane == 2, jnp.full((16,), i2_s), zeros))
    res[...] = vec
    pltpu.sync_copy(res, out_hbm.at[wid])


@jax.jit
def _bpr_partials(user0, item_i0, ratings, embed_user, embed_item):
    mesh = plsc.VectorSubcoreMesh(core_axis_name="c", subcore_axis_name="s")
    kfn = functools.partial(
        pl.kernel,
        out_type=jax.ShapeDtypeStruct((_NW, 16), jnp.float32),
        mesh=mesh,
        compiler_params=pltpu.CompilerParams(
            needs_layout_passes=False, use_tc_tiling_on_sc=False),
        scratch_types=[
            pltpu.VMEM((_BPW,), jnp.int32),
            pltpu.VMEM((_BPW,), jnp.int32),
            pltpu.VMEM((_BPW, _D), jnp.float32),
            pltpu.VMEM((_BPW, _D), jnp.float32),
            pltpu.VMEM((_BPW,), jnp.float32),
            pltpu.VMEM((16,), jnp.float32),
            pltpu.SemaphoreType.DMA,
        ],
    )(_tile_body)
    return kfn(user0, item_i0, ratings, embed_user, embed_item)


def kernel(user0, item_i0, ratings, embed_user, embed_item):
    parts = _bpr_partials(user0, item_i0, ratings, embed_user, embed_item)
    s = jnp.sum(parts, axis=0)
    loss2 = s[0]
    l2 = s[1] + s[2]
    return (loss2 + l2, loss2, l2)


# speedup vs baseline: 1.2372x; 1.0243x over previous
import functools

import jax
import jax.numpy as jnp
from jax import lax
from jax.experimental import pallas as pl
from jax.experimental.pallas import tpu as pltpu
from jax.experimental.pallas import tpu_sc as plsc

_LAMADA = 0.001
_B = 16384
_D = 64
_NC = 2
_NS = 16
_NW = _NC * _NS
_BPW = _B // _NW
_CHUNK = 128
_NCHUNK = _BPW // _CHUNK
_UNROLL = 16

_PARAMS = pltpu.CompilerParams(
    needs_layout_passes=False, use_tc_tiling_on_sc=False)
_MESH = dict(core_axis_name="c", subcore_axis_name="s")


def _wid():
    return lax.axis_index("s") * _NC + lax.axis_index("c")


def _gather_body(idx_hbm, table_hbm, sel_hbm, psum_hbm, idxv, rows, res, sem):
    wid = _wid()
    base = wid * _BPW
    pltpu.sync_copy(idx_hbm.at[pl.ds(base, _BPW)], idxv)
    copies = []
    for j in range(_NCHUNK):
        sl = pl.ds(j * _CHUNK, _CHUNK)
        copies.append(pltpu.async_copy(table_hbm.at[idxv.at[sl]], rows.at[sl], sem))
    for c in copies:
        c.wait()

    zeros = jnp.zeros((16,), jnp.float32)

    def row_block(blk, acc):
        for k in range(_UNROLL):
            r = blk * _UNROLL + k
            for c in range(_D // 16):
                v = rows[r, pl.ds(16 * c, 16)]
                acc = acc + v * v
        return acc

    sq = lax.fori_loop(0, _BPW // _UNROLL, row_block, zeros)
    sq_s = jnp.sum(sq) * (_LAMADA / (_B * _D))

    pltpu.sync_copy(rows, sel_hbm.at[pl.ds(base, _BPW), :])

    lane = lax.iota(jnp.int32, 16)
    res[...] = jnp.where(lane == 0, jnp.full((16,), sq_s), zeros)
    pltpu.sync_copy(res, psum_hbm.at[wid])


def _dot_body(ratings_hbm, usel_hbm, isel_hbm, psum_hbm,
              urows, irows, rat, res, sem):
    wid = _wid()
    base = wid * _BPW
    pltpu.sync_copy(ratings_hbm.at[pl.ds(base, _BPW)], rat)
    cu = pltpu.async_copy(usel_hbm.at[pl.ds(base, _BPW), :], urows, sem)
    ci = pltpu.async_copy(isel_hbm.at[pl.ds(base, _BPW), :], irows, sem)
    cu.wait()
    ci.wait()

    def row_block(blk, loss2_acc):
        rv = rat[pl.ds(blk * _UNROLL, 16)]
        for k in range(_UNROLL):
            r = blk * _UNROLL + k
            us = [urows[r, pl.ds(16 * c, 16)] for c in range(_D // 16)]
            vs = [irows[r, pl.ds(16 * c, 16)] for c in range(_D // 16)]
            t = us[0] * vs[0]
            for c in range(1, _D // 16):
                t = t + us[c] * vs[c]
            err = jnp.sum(t) - rv[k]
            loss2_acc = loss2_acc + err * err
        return loss2_acc

    loss2_s = lax.fori_loop(0, _BPW // _UNROLL, row_block, jnp.float32(0.0))
    loss2_s = loss2_s * (1.0 / _B)

    lane = lax.iota(jnp.int32, 16)
    zeros = jnp.zeros((16,), jnp.float32)
    res[...] = jnp.where(lane == 0, jnp.full((16,), loss2_s), zeros)
    pltpu.sync_copy(res, psum_hbm.at[wid])


def _gather_call(idx, table):
    kfn = functools.partial(
        pl.kernel,
        out_type=(jax.ShapeDtypeStruct((_B, _D), jnp.float32),
                  jax.ShapeDtypeStruct((_NW, 16), jnp.float32)),
        mesh=plsc.VectorSubcoreMesh(**_MESH),
        compiler_params=_PARAMS,
        scratch_types=[
            pltpu.VMEM((_BPW,), jnp.int32),
            pltpu.VMEM((_BPW, _D), jnp.float32),
            pltpu.VMEM((16,), jnp.float32),
            pltpu.SemaphoreType.DMA,
        ],
    )(_gather_body)
    return kfn(idx, table)


def _dot_call(ratings, usel, isel):
    kfn = functools.partial(
        pl.kernel,
        out_type=jax.ShapeDtypeStruct((_NW, 16), jnp.float32),
        mesh=plsc.VectorSubcoreMesh(**_MESH),
        compiler_params=_PARAMS,
        scratch_types=[
            pltpu.VMEM((_BPW, _D), jnp.float32),
            pltpu.VMEM((_BPW, _D), jnp.float32),
            pltpu.VMEM((_BPW,), jnp.float32),
            pltpu.VMEM((16,), jnp.float32),
            pltpu.SemaphoreType.DMA,
        ],
    )(_dot_body)
    return kfn(ratings, usel, isel)


def kernel(user0, item_i0, ratings, embed_user, embed_item):
    usel, up = _gather_call(user0, embed_user)
    isel, ip = _gather_call(item_i0, embed_item)
    dp = _dot_call(ratings, usel, isel)
    loss2 = jnp.sum(dp[:, 0])
    l2 = jnp.sum(up[:, 0]) + jnp.sum(ip[:, 0])
    return (loss2 + l2, loss2, l2)


# speedup vs baseline: 1.3090x; 1.0580x over previous
import functools

import jax
import jax.numpy as jnp
from jax import lax
from jax.experimental import pallas as pl
from jax.experimental.pallas import tpu as pltpu
from jax.experimental.pallas import tpu_sc as plsc

_LAMADA = 0.001
_B = 16384
_D = 64
_NC = 2
_NS = 16
_NW = _NC * _NS
_BPW = _B // _NW
_CHUNK = 128
_NCHUNK = _BPW // _CHUNK
_UNROLL = 16

_PARAMS = pltpu.CompilerParams(
    needs_layout_passes=False, use_tc_tiling_on_sc=False)
_MESH = dict(core_axis_name="c", subcore_axis_name="s")


def _wid():
    return lax.axis_index("s") * _NC + lax.axis_index("c")


def _gather_body(idx_hbm, table_hbm, sel_hbm, psum_hbm, idxv, rows, res, sem):
    wid = _wid()
    base = wid * _BPW
    pltpu.sync_copy(idx_hbm.at[pl.ds(base, _BPW)], idxv)
    copies = []
    for j in range(_NCHUNK):
        sl = pl.ds(j * _CHUNK, _CHUNK)
        copies.append(pltpu.async_copy(table_hbm.at[idxv.at[sl]], rows.at[sl], sem))
    for c in copies:
        c.wait()

    zeros = jnp.zeros((16,), jnp.float32)

    def row_block(blk, acc):
        for k in range(_UNROLL):
            r = blk * _UNROLL + k
            for c in range(_D // 16):
                v = rows[r, pl.ds(16 * c, 16)]
                acc = acc + v * v
        return acc

    sq = lax.fori_loop(0, _BPW // _UNROLL, row_block, zeros)
    sq_s = jnp.sum(sq) * (_LAMADA / (_B * _D))

    pltpu.sync_copy(rows, sel_hbm.at[pl.ds(base, _BPW), :])

    lane = lax.iota(jnp.int32, 16)
    res[...] = jnp.where(lane == 0, jnp.full((16,), sq_s), zeros)
    pltpu.sync_copy(res, psum_hbm.at[wid])


def _gather_dot_body(idx_hbm, ratings_hbm, table_hbm, usel_hbm, psum_hbm,
                     idxv, irows, urows, rat, res, sem):
    wid = _wid()
    base = wid * _BPW
    pltpu.sync_copy(idx_hbm.at[pl.ds(base, _BPW)], idxv)
    copies = [pltpu.async_copy(usel_hbm.at[pl.ds(base, _BPW), :], urows, sem)]
    for j in range(_NCHUNK):
        sl = pl.ds(j * _CHUNK, _CHUNK)
        copies.append(pltpu.async_copy(table_hbm.at[idxv.at[sl]], irows.at[sl], sem))
    pltpu.sync_copy(ratings_hbm.at[pl.ds(base, _BPW)], rat)
    for c in copies:
        c.wait()

    zeros = jnp.zeros((16,), jnp.float32)

    def row_block(blk, carry):
        loss2_acc, i2_acc = carry
        rv = rat[pl.ds(blk * _UNROLL, 16)]
        for k in range(_UNROLL):
            r = blk * _UNROLL + k
            us = [urows[r, pl.ds(16 * c, 16)] for c in range(_D // 16)]
            vs = [irows[r, pl.ds(16 * c, 16)] for c in range(_D // 16)]
            t = us[0] * vs[0]
            for c in range(1, _D // 16):
                t = t + us[c] * vs[c]
            err = jnp.sum(t) - rv[k]
            loss2_acc = loss2_acc + err * err
            for c in range(_D // 16):
                i2_acc = i2_acc + vs[c] * vs[c]
        return (loss2_acc, i2_acc)

    loss2_s, i2_acc = lax.fori_loop(
        0, _BPW // _UNROLL, row_block, (jnp.float32(0.0), zeros))
    loss2_s = loss2_s * (1.0 / _B)
    i2_s = jnp.sum(i2_acc) * (_LAMADA / (_B * _D))

    lane = lax.iota(jnp.int32, 16)
    res[...] = (jnp.where(lane == 0, jnp.full((16,), loss2_s), zeros)
                + jnp.where(lane == 1, jnp.full((16,), i2_s), zeros))
    pltpu.sync_copy(res, psum_hbm.at[wid])


def _gather_call(idx, table):
    kfn = functools.partial(
        pl.kernel,
        out_type=(jax.ShapeDtypeStruct((_B, _D), jnp.float32),
                  jax.ShapeDtypeStruct((_NW, 16), jnp.float32)),
        mesh=plsc.VectorSubcoreMesh(**_MESH),
        compiler_params=_PARAMS,
        scratch_types=[
            pltpu.VMEM((_BPW,), jnp.int32),
            pltpu.VMEM((_BPW, _D), jnp.float32),
            pltpu.VMEM((16,), jnp.float32),
            pltpu.SemaphoreType.DMA,
        ],
    )(_gather_body)
    return kfn(idx, table)


def _gather_dot_call(item_i0, ratings, embed_item, usel):
    kfn = functools.partial(
        pl.kernel,
        out_type=jax.ShapeDtypeStruct((_NW, 16), jnp.float32),
        mesh=plsc.VectorSubcoreMesh(**_MESH),
        compiler_params=_PARAMS,
        scratch_types=[
            pltpu.VMEM((_BPW,), jnp.int32),
            pltpu.VMEM((_BPW, _D), jnp.float32),
            pltpu.VMEM((_BPW, _D), jnp.float32),
            pltpu.VMEM((_BPW,), jnp.float32),
            pltpu.VMEM((16,), jnp.float32),
            pltpu.SemaphoreType.DMA,
        ],
    )(_gather_dot_body)
    return kfn(item_i0, ratings, embed_item, usel)


def kernel(user0, item_i0, ratings, embed_user, embed_item):
    usel, up = _gather_call(user0, embed_user)
    dp = _gather_dot_call(item_i0, ratings, embed_item, usel)
    loss2 = jnp.sum(dp[:, 0])
    l2 = jnp.sum(up[:, 0]) + jnp.sum(dp[:, 1])
    return (loss2 + l2, loss2, l2)
